# Initial kernel scaffold; baseline (speedup 1.0000x reference)
import functools

import jax
import jax.numpy as jnp
from jax import lax
from jax.experimental import pallas as pl
from jax.experimental.pallas import tpu as pltpu
from jax.experimental.pallas import tpu_sc as plsc

N = 10000
E = 320000
D = 128
UNITS = 32
HEADS = 4
HID = UNITS * HEADS
OUT = 2

NC = 2
NS = 16
LANES = 16
NW = NC * NS

N_PAD = 10240
ACCW = 144
CHUNK = 128
E_PAD = 327680
EP_TILE = E_PAD // NW
NCHUNK = EP_TILE // CHUNK
ROWS_PER_TILE = N_PAD // NS

BR = 1280
GRID = N_PAD // BR

_PREC = jax.lax.Precision.HIGHEST


def _dot(a, b):
    return jnp.dot(a, b, precision=_PREC, preferred_element_type=jnp.float32)


def _tc_front(x, W0, b0, W1, b1, K0, M0):
    def body(x_ref, w0_ref, b0_ref, w1_ref, b1_ref, k_ref, m_ref,
             h_ref, hk_ref, st_ref):
        hh = jnp.maximum(_dot(x_ref[...], w0_ref[...]) + b0_ref[...], 0.0)
        hh = jnp.maximum(_dot(hh, w1_ref[...]) + b1_ref[...], 0.0)
        h_ref[...] = hh
        hk = _dot(hh, k_ref[...])
        hk_ref[...] = hk
        st_ref[...] = _dot(hk, m_ref[...])

    full = lambda shape: pl.BlockSpec(shape, lambda i: (0, 0))
    return pl.pallas_call(
        body,
        grid=(GRID,),
        in_specs=[
            pl.BlockSpec((BR, D), lambda i: (i, 0)),
            full((D, HID)), full((1, HID)), full((HID, HID)), full((1, HID)),
            full((HID, HID)), full((HID, 16)),
        ],
        out_specs=[
            pl.BlockSpec((BR, HID), lambda i: (i, 0)),
            pl.BlockSpec((BR, HID), lambda i: (i, 0)),
            pl.BlockSpec((BR, 16), lambda i: (i, 0)),
        ],
        out_shape=[
            jax.ShapeDtypeStruct((N_PAD, HID), jnp.float32),
            jax.ShapeDtypeStruct((N_PAD, HID), jnp.float32),
            jax.ShapeDtypeStruct((N_PAD, 16), jnp.float32),
        ],
    )(x, W0, b0, W1, b1, K0, M0)


def _tc_mid(wA, wB, dA, dB, hprev, K, M, S):
    def body(wa_ref, wb_ref, da_ref, db_ref, hp_ref, k_ref, m_ref, s_ref,
             h_ref, hk_ref, st_ref):
        w = wa_ref[...] + wb_ref[...]
        den = da_ref[...] + db_ref[...]
        den128 = _dot(den, s_ref[...]) + 1e-7
        h1 = jnp.maximum(w / den128, 0.0) + hp_ref[...]
        h_ref[...] = h1
        hk = _dot(h1, k_ref[...])
        hk_ref[...] = hk
        st_ref[...] = _dot(hk, m_ref[...])

    full = lambda shape: pl.BlockSpec(shape, lambda i: (0, 0))
    row = lambda w: pl.BlockSpec((BR, w), lambda i: (i, 0))
    return pl.pallas_call(
        body,
        grid=(GRID,),
        in_specs=[row(HID), row(HID), row(4), row(4), row(HID),
                  full((HID, HID)), full((HID, 16)), full((HEADS, HID))],
        out_specs=[row(HID), row(HID), row(16)],
        out_shape=[
            jax.ShapeDtypeStruct((N_PAD, HID), jnp.float32),
            jax.ShapeDtypeStruct((N_PAD, HID), jnp.float32),
            jax.ShapeDtypeStruct((N_PAD, 16), jnp.float32),
        ],
    )(wA, wB, dA, dB, hprev, K, M, S)


def _tc_out(wA, wB, dA, dB, hprev, Wo, bo, S):
    def body(wa_ref, wb_ref, da_ref, db_ref, hp_ref, wo_ref, bo_ref, s_ref,
             y_ref):
        w = wa_ref[...] + wb_ref[...]
        den = da_ref[...] + db_ref[...]
        den128 = _dot(den, s_ref[...]) + 1e-7
        h2 = jnp.maximum(w / den128, 0.0) + hp_ref[...]
        y_ref[...] = _dot(h2, wo_ref[...]) + bo_ref[...]

    full = lambda shape: pl.BlockSpec(shape, lambda i: (0, 0))
    row = lambda w: pl.BlockSpec((BR, w), lambda i: (i, 0))
    return pl.pallas_call(
        body,
        grid=(GRID,),
        in_specs=[row(HID), row(HID), row(4), row(4), row(HID),
                  full((HID, OUT)), full((1, OUT)), full((HEADS, HID))],
        out_specs=row(OUT),
        out_shape=jax.ShapeDtypeStruct((N_PAD, OUT), jnp.float32),
    )(wA, wB, dA, dB, hprev, Wo, bo, S)


def _sc_gat_layer(hk, sctab, tgt, src):
    mesh = plsc.VectorSubcoreMesh(core_axis_name="c", subcore_axis_name="s")

    @functools.partial(
        pl.kernel,
        out_type=jax.ShapeDtypeStruct((NC, N_PAD, ACCW), jnp.float32),
        mesh=mesh,
        scratch_types=[
            pltpu.VMEM((CHUNK,), jnp.int32),
            pltpu.VMEM((CHUNK,), jnp.int32),
            pltpu.VMEM((CHUNK, 16), jnp.float32),
            pltpu.VMEM((CHUNK, 16), jnp.float32),
            pltpu.VMEM((CHUNK, HID), jnp.float32),
            pltpu.VMEM((CHUNK, ACCW), jnp.float32),
            pltpu.VMEM_SHARED((N_PAD, ACCW), jnp.float32),
            pltpu.SemaphoreType.DMA,
            pltpu.SemaphoreType.DMA,
            pltpu.SemaphoreType.DMA,
        ],
    )
    def k(hk_hbm, sctab_hbm, tgt_hbm, src_hbm, out_hbm,
          tgt_v, src_v, a_v, b_v, rows_v, orow_v, acc_sh, sem0, sem1, sem2):
        cid = lax.axis_index("c")
        sid = lax.axis_index("s")
        wid = cid * NS + sid

        zero16 = jnp.zeros((LANES,), jnp.float32)

        @pl.loop(0, CHUNK)
        def _zero_rows(i):
            @pl.loop(0, ACCW, step=LANES)
            def _zero_cols(j):
                orow_v[i, pl.ds(j, LANES)] = zero16

        row0 = sid * ROWS_PER_TILE
        for off in range(0, ROWS_PER_TILE, CHUNK):
            pltpu.sync_copy(orow_v, acc_sh.at[pl.ds(row0 + off, CHUNK)])
        plsc.subcore_barrier()

        iota = lax.iota(jnp.int32, LANES)
        shift4 = jnp.minimum(iota + HEADS, LANES - 1)
        headmask = iota < HEADS

        ebase = wid * EP_TILE

        @pl.loop(0, NCHUNK)
        def _chunk(ci):
            base = ebase + ci * CHUNK
            pltpu.sync_copy(tgt_hbm.at[pl.ds(base, CHUNK)], tgt_v)
            pltpu.sync_copy(src_hbm.at[pl.ds(base, CHUNK)], src_v)
            cp0 = pltpu.async_copy(sctab_hbm.at[tgt_v], a_v, sem0)
            cp1 = pltpu.async_copy(sctab_hbm.at[src_v], b_v, sem1)
            cp2 = pltpu.async_copy(hk_hbm.at[src_v], rows_v, sem2)
            cp0.wait()
            cp1.wait()
            cp2.wait()

            @pl.loop(0, CHUNK)
            def _edge(e):
                erow = jnp.full((LANES,), e, jnp.int32)
                va = a_v[e, :]
                vb = plsc.load_gather(b_v, [erow, shift4])
                s = va + vb
                s = jnp.maximum(s, 0.2 * s)
                s = jnp.clip(s, -2.0, 2.0)
                p = jnp.exp(s)
                p = jnp.where(headmask, p, 0.0)
                orow_v[e, pl.ds(HID, LANES)] = p
                for h in range(HEADS):
                    m = plsc.load_gather(
                        orow_v, [erow, jnp.full((LANES,), HID + h, jnp.int32)])
                    c0 = UNITS * h
                    orow_v[e, pl.ds(c0, LANES)] = rows_v[e, pl.ds(c0, LANES)] * m
                    orow_v[e, pl.ds(c0 + LANES, LANES)] = (
                        rows_v[e, pl.ds(c0 + LANES, LANES)] * m)

            pltpu.sync_copy(orow_v, acc_sh.at[tgt_v], add=True)

        plsc.subcore_barrier()
        for off in range(0, ROWS_PER_TILE, CHUNK):
            pltpu.sync_copy(acc_sh.at[pl.ds(row0 + off, CHUNK)],
                            out_hbm.at[cid, pl.ds(row0 + off, CHUNK)])

    return k(hk, sctab, tgt, src)


def _make_M(att_l):
    at_w = att_l[:, :UNITS, 0]
    as_w = att_l[:, UNITS:, 0]
    eye = jnp.eye(HEADS, dtype=jnp.float32)
    Mt = jnp.einsum("hu,hk->huk", at_w, eye).reshape(HID, HEADS)
    Ms = jnp.einsum("hu,hk->huk", as_w, eye).reshape(HID, HEADS)
    return jnp.concatenate([Mt, Ms, jnp.zeros((HID, 8), jnp.float32)], axis=1)


def kernel(x, edges, W0, b0, W1, b1, gat_kernels, gat_attn, Wo, bo):
    K0 = gat_kernels[0].transpose(1, 0, 2).reshape(HID, HID)
    K1 = gat_kernels[1].transpose(1, 0, 2).reshape(HID, HID)
    M0 = _make_M(gat_attn[0])
    M1 = _make_M(gat_attn[1])
    S = jnp.repeat(jnp.eye(HEADS, dtype=jnp.float32), UNITS, axis=1)

    xp = jnp.zeros((N_PAD, D), jnp.float32).at[:N].set(x)
    padi = jnp.full((E_PAD - E,), N, jnp.int32)
    tgt = jnp.concatenate([edges[:, 0], padi])
    src = jnp.concatenate([edges[:, 1], padi])

    h0, hk0, st0 = _tc_front(xp, W0, b0.reshape(1, -1), W1, b1.reshape(1, -1),
                             K0, M0)
    acc0 = _sc_gat_layer(hk0, st0, tgt, src)
    h1, hk1, st1 = _tc_mid(acc0[0, :, :HID], acc0[1, :, :HID],
                           acc0[0, :, HID:HID + 4], acc0[1, :, HID:HID + 4],
                           h0, K1, M1, S)
    acc1 = _sc_gat_layer(hk1, st1, tgt, src)
    y = _tc_out(acc1[0, :, :HID], acc1[1, :, :HID],
                acc1[0, :, HID:HID + 4], acc1[1, :, HID:HID + 4],
                h1, Wo, bo.reshape(1, -1), S)
    return y[:N]


# speedup vs baseline: 9.8007x; 9.8007x over previous
import dataclasses
import functools

import jax
import jax.numpy as jnp
from jax import lax
from jax.experimental import pallas as pl
from jax.experimental.pallas import tpu as pltpu
from jax.experimental.pallas import tpu_sc as plsc

N = 10000
E = 320000
D = 128
UNITS = 32
HEADS = 4
HID = UNITS * HEADS
OUT = 2

NC = 2
NS = 16
LANES = 16
NW = NC * NS

N_PAD = 10176
ACCW = 144
CHUNK = 128
E_PAD = 327680
EP_TILE = E_PAD // NW
NCHUNK = EP_TILE // CHUNK
ROWS_PER_TILE = N_PAD // NS

BR = 1272
GRID = N_PAD // BR

_PREC = jax.lax.Precision.HIGHEST


def _dot(a, b):
    return jnp.dot(a, b, precision=_PREC, preferred_element_type=jnp.float32)


def _tc_front(x, W0, b0, W1, b1, K0, M0):
    def body(x_ref, w0_ref, b0_ref, w1_ref, b1_ref, k_ref, m_ref,
             h_ref, hk_ref, st_ref):
        hh = jnp.maximum(_dot(x_ref[...], w0_ref[...]) + b0_ref[...], 0.0)
        hh = jnp.maximum(_dot(hh, w1_ref[...]) + b1_ref[...], 0.0)
        h_ref[...] = hh
        hk = _dot(hh, k_ref[...])
        hk_ref[...] = hk
        st_ref[...] = _dot(hk, m_ref[...])

    full = lambda shape: pl.BlockSpec(shape, lambda i: (0, 0))
    return pl.pallas_call(
        body,
        grid=(GRID,),
        in_specs=[
            pl.BlockSpec((BR, D), lambda i: (i, 0)),
            full((D, HID)), full((1, HID)), full((HID, HID)), full((1, HID)),
            full((HID, HID)), full((HID, 16)),
        ],
        out_specs=[
            pl.BlockSpec((BR, HID), lambda i: (i, 0)),
            pl.BlockSpec((BR, HID), lambda i: (i, 0)),
            pl.BlockSpec((BR, 16), lambda i: (i, 0)),
        ],
        out_shape=[
            jax.ShapeDtypeStruct((N_PAD, HID), jnp.float32),
            jax.ShapeDtypeStruct((N_PAD, HID), jnp.float32),
            jax.ShapeDtypeStruct((N_PAD, 16), jnp.float32),
        ],
    )(x, W0, b0, W1, b1, K0, M0)


def _tc_mid(wA, wB, dA, dB, hprev, K, M, S):
    def body(wa_ref, wb_ref, da_ref, db_ref, hp_ref, k_ref, m_ref, s_ref,
             h_ref, hk_ref, st_ref):
        w = wa_ref[...] + wb_ref[...]
        den = da_ref[...] + db_ref[...]
        den128 = _dot(den, s_ref[...]) + 1e-7
        h1 = jnp.maximum(w / den128, 0.0) + hp_ref[...]
        h_ref[...] = h1
        hk = _dot(h1, k_ref[...])
        hk_ref[...] = hk
        st_ref[...] = _dot(hk, m_ref[...])

    full = lambda shape: pl.BlockSpec(shape, lambda i: (0, 0))
    row = lambda w: pl.BlockSpec((BR, w), lambda i: (i, 0))
    return pl.pallas_call(
        body,
        grid=(GRID,),
        in_specs=[row(HID), row(HID), row(4), row(4), row(HID),
                  full((HID, HID)), full((HID, 16)), full((HEADS, HID))],
        out_specs=[row(HID), row(HID), row(16)],
        out_shape=[
            jax.ShapeDtypeStruct((N_PAD, HID), jnp.float32),
            jax.ShapeDtypeStruct((N_PAD, HID), jnp.float32),
            jax.ShapeDtypeStruct((N_PAD, 16), jnp.float32),
        ],
    )(wA, wB, dA, dB, hprev, K, M, S)


def _tc_out(wA, wB, dA, dB, hprev, Wo, bo, S):
    def body(wa_ref, wb_ref, da_ref, db_ref, hp_ref, wo_ref, bo_ref, s_ref,
             y_ref):
        w = wa_ref[...] + wb_ref[...]
        den = da_ref[...] + db_ref[...]
        den128 = _dot(den, s_ref[...]) + 1e-7
        h2 = jnp.maximum(w / den128, 0.0) + hp_ref[...]
        y_ref[...] = _dot(h2, wo_ref[...]) + bo_ref[...]

    full = lambda shape: pl.BlockSpec(shape, lambda i: (0, 0))
    row = lambda w: pl.BlockSpec((BR, w), lambda i: (i, 0))
    return pl.pallas_call(
        body,
        grid=(GRID,),
        in_specs=[row(HID), row(HID), row(4), row(4), row(HID),
                  full((HID, OUT)), full((1, OUT)), full((HEADS, HID))],
        out_specs=row(OUT),
        out_shape=jax.ShapeDtypeStruct((N_PAD, OUT), jnp.float32),
    )(wA, wB, dA, dB, hprev, Wo, bo, S)


def _sc_gat_layer(hk, sctab, tgt, src):
    mesh = plsc.VectorSubcoreMesh(core_axis_name="c", subcore_axis_name="s")
    cp = pltpu.CompilerParams()
    if "needs_layout_passes" in pltpu.CompilerParams.__dataclass_fields__:
        cp = dataclasses.replace(cp, needs_layout_passes=False)
    if "use_tc_tiling_on_sc" in pltpu.CompilerParams.__dataclass_fields__:
        cp = dataclasses.replace(cp, use_tc_tiling_on_sc=False)

    @functools.partial(
        pl.kernel,
        out_type=jax.ShapeDtypeStruct((NC, N_PAD, ACCW), jnp.float32),
        mesh=mesh,
        compiler_params=cp,
        scratch_types=[
            pltpu.VMEM((CHUNK,), jnp.int32),
            pltpu.VMEM((CHUNK,), jnp.int32),
            pltpu.VMEM((CHUNK, 16), jnp.float32),
            pltpu.VMEM((CHUNK, 16), jnp.float32),
            pltpu.VMEM((CHUNK, HID), jnp.float32),
            pltpu.VMEM((CHUNK, ACCW), jnp.float32),
            pltpu.VMEM_SHARED((N_PAD, ACCW), jnp.float32),
            pltpu.SemaphoreType.DMA,
            pltpu.SemaphoreType.DMA,
            pltpu.SemaphoreType.DMA,
        ],
    )
    def k(hk_hbm, sctab_hbm, tgt_hbm, src_hbm, out_hbm,
          tgt_v, src_v, a_v, b_v, rows_v, orow_v, acc_sh, sem0, sem1, sem2):
        cid = lax.axis_index("c")
        sid = lax.axis_index("s")
        wid = cid * NS + sid

        zero16 = jnp.zeros((LANES,), jnp.float32)

        @pl.loop(0, CHUNK)
        def _zero_rows(i):
            @pl.loop(0, ACCW, step=LANES)
            def _zero_cols(j):
                orow_v[i, pl.ds(j, LANES)] = zero16

        row0 = sid * ROWS_PER_TILE
        for off in range(0, ROWS_PER_TILE, CHUNK):
            sz = min(CHUNK, ROWS_PER_TILE - off)
            pltpu.sync_copy(orow_v.at[pl.ds(0, sz)],
                            acc_sh.at[pl.ds(row0 + off, sz)])
        plsc.subcore_barrier()

        iota = lax.iota(jnp.int32, LANES)
        shift4 = jnp.minimum(iota + HEADS, LANES - 1)
        headmask = iota < HEADS

        ebase = wid * EP_TILE

        @pl.loop(0, NCHUNK)
        def _chunk(ci):
            base = ebase + ci * CHUNK
            pltpu.sync_copy(tgt_hbm.at[pl.ds(base, CHUNK)], tgt_v)
            pltpu.sync_copy(src_hbm.at[pl.ds(base, CHUNK)], src_v)
            cp0 = pltpu.async_copy(sctab_hbm.at[tgt_v], a_v, sem0)
            cp1 = pltpu.async_copy(sctab_hbm.at[src_v], b_v, sem1)
            cp2 = pltpu.async_copy(hk_hbm.at[src_v], rows_v, sem2)
            cp0.wait()
            cp1.wait()
            cp2.wait()

            @pl.loop(0, CHUNK)
            def _edge(e):
                erow = jnp.full((LANES,), e, jnp.int32)
                va = a_v[e, :]
                vb = plsc.load_gather(b_v, [erow, shift4])
                s = va + vb
                s = jnp.maximum(s, 0.2 * s)
                s = jnp.clip(s, -2.0, 2.0)
                p = jnp.exp(s)
                p = jnp.where(headmask, p, 0.0)
                orow_v[e, pl.ds(HID, LANES)] = p
                for h in range(HEADS):
                    m = plsc.load_gather(
                        orow_v, [erow, jnp.full((LANES,), HID + h, jnp.int32)])
                    c0 = UNITS * h
                    orow_v[e, pl.ds(c0, LANES)] = rows_v[e, pl.ds(c0, LANES)] * m
                    orow_v[e, pl.ds(c0 + LANES, LANES)] = (
                        rows_v[e, pl.ds(c0 + LANES, LANES)] * m)

            pltpu.sync_copy(orow_v, acc_sh.at[tgt_v], add=True)

        plsc.subcore_barrier()
        for off in range(0, ROWS_PER_TILE, CHUNK):
            sz = min(CHUNK, ROWS_PER_TILE - off)
            pltpu.sync_copy(acc_sh.at[pl.ds(row0 + off, sz)],
                            out_hbm.at[cid, pl.ds(row0 + off, sz)])

    return k(hk, sctab, tgt, src)


def _make_M(att_l):
    at_w = att_l[:, :UNITS, 0]
    as_w = att_l[:, UNITS:, 0]
    eye = jnp.eye(HEADS, dtype=jnp.float32)
    Mt = jnp.einsum("hu,hk->huk", at_w, eye).reshape(HID, HEADS)
    Ms = jnp.einsum("hu,hk->huk", as_w, eye).reshape(HID, HEADS)
    return jnp.concatenate([Mt, Ms, jnp.zeros((HID, 8), jnp.float32)], axis=1)


def kernel(x, edges, W0, b0, W1, b1, gat_kernels, gat_attn, Wo, bo):
    K0 = gat_kernels[0].transpose(1, 0, 2).reshape(HID, HID)
    K1 = gat_kernels[1].transpose(1, 0, 2).reshape(HID, HID)
    M0 = _make_M(gat_attn[0])
    M1 = _make_M(gat_attn[1])
    S = jnp.repeat(jnp.eye(HEADS, dtype=jnp.float32), UNITS, axis=1)

    xp = jnp.zeros((N_PAD, D), jnp.float32).at[:N].set(x)
    padi = jnp.full((E_PAD - E,), N, jnp.int32)
    tgt = jnp.concatenate([edges[:, 0], padi])
    src = jnp.concatenate([edges[:, 1], padi])

    h0, hk0, st0 = _tc_front(xp, W0, b0.reshape(1, -1), W1, b1.reshape(1, -1),
                             K0, M0)
    acc0 = _sc_gat_layer(hk0, st0, tgt, src)
    h1, hk1, st1 = _tc_mid(acc0[0, :, :HID], acc0[1, :, :HID],
                           acc0[0, :, HID:HID + 4], acc0[1, :, HID:HID + 4],
                           h0, K1, M1, S)
    acc1 = _sc_gat_layer(hk1, st1, tgt, src)
    y = _tc_out(acc1[0, :, :HID], acc1[1, :, :HID],
                acc1[0, :, HID:HID + 4], acc1[1, :, HID:HID + 4],
                h1, Wo, bo.reshape(1, -1), S)
    return y[:N]


# speedup vs baseline: 19.8279x; 2.0231x over previous
import dataclasses
import functools

import jax
import jax.numpy as jnp
from jax import lax
from jax.experimental import pallas as pl
from jax.experimental.pallas import tpu as pltpu
from jax.experimental.pallas import tpu_sc as plsc

N = 10000
E = 320000
D = 128
UNITS = 32
HEADS = 4
HID = UNITS * HEADS
OUT = 2

NC = 2
NS = 16
LANES = 16
NW = NC * NS

N_PAD = 10016
ACCW = 144
CHUNK = 64
E_PAD = 327680
EP_TILE = E_PAD // NW
NCHUNK = EP_TILE // CHUNK
ROWS_PER_TILE = N_PAD // NS

BR = 2504
GRID = N_PAD // BR

_PREC = jax.lax.Precision.HIGHEST


def _dot(a, b):
    return jnp.dot(a, b, precision=_PREC, preferred_element_type=jnp.float32)


def _tc_front(x, W0, b0, W1, b1, K0, M0):
    def body(x_ref, w0_ref, b0_ref, w1_ref, b1_ref, k_ref, m_ref,
             h_ref, hk_ref, st_ref):
        hh = jnp.maximum(_dot(x_ref[...], w0_ref[...]) + b0_ref[...], 0.0)
        hh = jnp.maximum(_dot(hh, w1_ref[...]) + b1_ref[...], 0.0)
        h_ref[...] = hh
        hk = _dot(hh, k_ref[...])
        hk_ref[...] = hk
        st_ref[...] = _dot(hk, m_ref[...])

    full = lambda shape: pl.BlockSpec(shape, lambda i: (0, 0))
    return pl.pallas_call(
        body,
        grid=(GRID,),
        in_specs=[
            pl.BlockSpec((BR, D), lambda i: (i, 0)),
            full((D, HID)), full((1, HID)), full((HID, HID)), full((1, HID)),
            full((HID, HID)), full((HID, 16)),
        ],
        out_specs=[
            pl.BlockSpec((BR, HID), lambda i: (i, 0)),
            pl.BlockSpec((BR, HID), lambda i: (i, 0)),
            pl.BlockSpec((BR, 16), lambda i: (i, 0)),
        ],
        out_shape=[
            jax.ShapeDtypeStruct((N_PAD, HID), jnp.float32),
            jax.ShapeDtypeStruct((N_PAD, HID), jnp.float32),
            jax.ShapeDtypeStruct((N_PAD, 16), jnp.float32),
        ],
    )(x, W0, b0, W1, b1, K0, M0)


def _tc_mid(wA, wB, dA, dB, hprev, K, M, S):
    def body(wa_ref, wb_ref, da_ref, db_ref, hp_ref, k_ref, m_ref, s_ref,
             h_ref, hk_ref, st_ref):
        w = wa_ref[...] + wb_ref[...]
        den = da_ref[...] + db_ref[...]
        den128 = _dot(den, s_ref[...]) + 1e-7
        h1 = jnp.maximum(w / den128, 0.0) + hp_ref[...]
        h_ref[...] = h1
        hk = _dot(h1, k_ref[...])
        hk_ref[...] = hk
        st_ref[...] = _dot(hk, m_ref[...])

    full = lambda shape: pl.BlockSpec(shape, lambda i: (0, 0))
    row = lambda w: pl.BlockSpec((BR, w), lambda i: (i, 0))
    return pl.pallas_call(
        body,
        grid=(GRID,),
        in_specs=[row(HID), row(HID), row(4), row(4), row(HID),
                  full((HID, HID)), full((HID, 16)), full((HEADS, HID))],
        out_specs=[row(HID), row(HID), row(16)],
        out_shape=[
            jax.ShapeDtypeStruct((N_PAD, HID), jnp.float32),
            jax.ShapeDtypeStruct((N_PAD, HID), jnp.float32),
            jax.ShapeDtypeStruct((N_PAD, 16), jnp.float32),
        ],
    )(wA, wB, dA, dB, hprev, K, M, S)


def _tc_out(wA, wB, dA, dB, hprev, Wo, bo, S):
    def body(wa_ref, wb_ref, da_ref, db_ref, hp_ref, wo_ref, bo_ref, s_ref,
             y_ref):
        w = wa_ref[...] + wb_ref[...]
        den = da_ref[...] + db_ref[...]
        den128 = _dot(den, s_ref[...]) + 1e-7
        h2 = jnp.maximum(w / den128, 0.0) + hp_ref[...]
        y_ref[...] = _dot(h2, wo_ref[...]) + bo_ref[...]

    full = lambda shape: pl.BlockSpec(shape, lambda i: (0, 0))
    row = lambda w: pl.BlockSpec((BR, w), lambda i: (i, 0))
    return pl.pallas_call(
        body,
        grid=(GRID,),
        in_specs=[row(HID), row(HID), row(4), row(4), row(HID),
                  full((HID, OUT)), full((1, OUT)), full((HEADS, HID))],
        out_specs=row(OUT),
        out_shape=jax.ShapeDtypeStruct((N_PAD, OUT), jnp.float32),
    )(wA, wB, dA, dB, hprev, Wo, bo, S)


def _sc_gat_layer(hk, sctab, tgt, src):
    mesh = plsc.VectorSubcoreMesh(core_axis_name="c", subcore_axis_name="s")
    cp = pltpu.CompilerParams()
    if "needs_layout_passes" in pltpu.CompilerParams.__dataclass_fields__:
        cp = dataclasses.replace(cp, needs_layout_passes=False)
    if "use_tc_tiling_on_sc" in pltpu.CompilerParams.__dataclass_fields__:
        cp = dataclasses.replace(cp, use_tc_tiling_on_sc=False)

    scratch_types=(
        [pltpu.VMEM((CHUNK,), jnp.int32) for _ in range(4)]
        + [pltpu.VMEM((CHUNK,), jnp.int32) for _ in range(4)]
        + [pltpu.VMEM((CHUNK, 16), jnp.float32) for _ in range(2)]
        + [pltpu.VMEM((CHUNK, 16), jnp.float32) for _ in range(2)]
        + [pltpu.VMEM((CHUNK, HID), jnp.float32) for _ in range(2)]
        + [pltpu.VMEM((CHUNK, ACCW), jnp.float32) for _ in range(2)]
        + [pltpu.VMEM_SHARED((N_PAD, ACCW), jnp.float32)]
        + [pltpu.SemaphoreType.DMA for _ in range(8)]
    )

    @functools.partial(
        pl.kernel,
        out_type=jax.ShapeDtypeStruct((NC, N_PAD, ACCW), jnp.float32),
        mesh=mesh,
        compiler_params=cp,
        scratch_types=scratch_types,
    )
    def k(hk_hbm, sctab_hbm, tgt_hbm, src_hbm, out_hbm,
          t0, t1, t2, t3, s0, s1, s2, s3, a0, a1, b0, b1, r0, r1, o0, o1,
          acc_sh, i_sem0, i_sem1, i_sem2, i_sem3, gsem0, gsem1, ssem0, ssem1):
        cid = lax.axis_index("c")
        sid = lax.axis_index("s")
        wid = cid * NS + sid
        tbuf, sbuf = (t0, t1, t2, t3), (s0, s1, s2, s3)
        abuf, bbuf, rbuf, obuf = (a0, a1), (b0, b1), (r0, r1), (o0, o1)
        isem = (i_sem0, i_sem1, i_sem2, i_sem3)
        gsem, ssem = (gsem0, gsem1), (ssem0, ssem1)

        zero16 = jnp.zeros((LANES,), jnp.float32)

        for o in obuf:
            @pl.loop(0, CHUNK)
            def _zero_rows(i):
                @pl.loop(0, ACCW, step=LANES)
                def _zero_cols(j):
                    o[i, pl.ds(j, LANES)] = zero16

        row0 = sid * ROWS_PER_TILE
        for off in range(0, ROWS_PER_TILE, CHUNK):
            sz = min(CHUNK, ROWS_PER_TILE - off)
            pltpu.sync_copy(o0.at[pl.ds(0, sz)],
                            acc_sh.at[pl.ds(row0 + off, sz)])
        plsc.subcore_barrier()

        iota = lax.iota(jnp.int32, LANES)
        idiv4 = lax.shift_right_logical(iota, 2)
        imod4 = iota & 3
        col_a = imod4
        col_b = imod4 + HEADS
        col_p = imod4 + HID
        midx = [jnp.full((LANES,), j, jnp.int32) for j in range(LANES)]

        ebase = wid * EP_TILE

        def start_idx(ci, ib):
            base = ebase + ci * CHUNK
            pltpu.async_copy(tgt_hbm.at[pl.ds(base, CHUNK)], tbuf[ib], isem[ib])
            pltpu.async_copy(src_hbm.at[pl.ds(base, CHUNK)], sbuf[ib], isem[ib])

        def wait_idx(ci, ib):
            base = ebase + ci * CHUNK
            pltpu.make_async_copy(tgt_hbm.at[pl.ds(base, CHUNK)], tbuf[ib],
                                  isem[ib]).wait()
            pltpu.make_async_copy(src_hbm.at[pl.ds(base, CHUNK)], sbuf[ib],
                                  isem[ib]).wait()

        def start_gathers(gb, ib):
            pltpu.async_copy(sctab_hbm.at[tbuf[ib]], abuf[gb], gsem[gb])
            pltpu.async_copy(sctab_hbm.at[sbuf[ib]], bbuf[gb], gsem[gb])
            pltpu.async_copy(hk_hbm.at[sbuf[ib]], rbuf[gb], gsem[gb])

        def wait_gathers(gb, ib):
            pltpu.make_async_copy(sctab_hbm.at[tbuf[ib]], abuf[gb],
                                  gsem[gb]).wait()
            pltpu.make_async_copy(sctab_hbm.at[sbuf[ib]], bbuf[gb],
                                  gsem[gb]).wait()
            pltpu.make_async_copy(hk_hbm.at[sbuf[ib]], rbuf[gb],
                                  gsem[gb]).wait()

        def wait_scatter(gb, ib):
            pltpu.make_async_copy(obuf[gb], acc_sh.at[tbuf[ib]],
                                  ssem[gb]).wait()

        def compute_chunk(a_v, b_v, rows_v, orow_v):
            @pl.loop(0, CHUNK // 4)
            def _group(g):
                e0 = g * 4
                rowidx = idiv4 + e0
                va = plsc.load_gather(a_v, [rowidx, col_a])
                vb = plsc.load_gather(b_v, [rowidx, col_b])
                s = va + vb
                s = jnp.maximum(s, 0.2 * s)
                s = jnp.clip(s, -2.0, 2.0)
                p4 = jnp.exp(s)
                plsc.store_scatter(orow_v, [rowidx, col_p], p4)
                for kk in range(4):
                    e = e0 + kk
                    for h in range(HEADS):
                        m = jnp.take_along_axis(p4, midx[4 * kk + h], axis=0,
                                                mode="promise_in_bounds")
                        c0 = UNITS * h
                        orow_v[e, pl.ds(c0, LANES)] = (
                            rows_v[e, pl.ds(c0, LANES)] * m)
                        orow_v[e, pl.ds(c0 + LANES, LANES)] = (
                            rows_v[e, pl.ds(c0 + LANES, LANES)] * m)

        start_idx(0, 0)
        start_idx(1, 1)
        wait_idx(0, 0)
        start_gathers(0, 0)

        @pl.loop(0, NCHUNK, step=4)
        def _quad(ci):
            for b in range(4):
                cur = ci + b
                gb = b % 2

                @pl.when(cur >= 2)
                def _ws():
                    wait_scatter(gb, (b - 2) % 4)

                @pl.when(cur + 2 < NCHUNK)
                def _pi():
                    start_idx(cur + 2, (b + 2) % 4)

                @pl.when(cur + 1 < NCHUNK)
                def _pg():
                    wait_idx(cur + 1, (b + 1) % 4)
                    start_gathers(1 - gb, (b + 1) % 4)

                wait_gathers(gb, b)
                compute_chunk(abuf[gb], bbuf[gb], rbuf[gb], obuf[gb])
                pltpu.async_copy(obuf[gb], acc_sh.at[tbuf[b]], ssem[gb],
                                 add=True)

        wait_scatter(0, 2)
        wait_scatter(1, 3)
        plsc.subcore_barrier()
        for off in range(0, ROWS_PER_TILE, CHUNK):
            sz = min(CHUNK, ROWS_PER_TILE - off)
            pltpu.sync_copy(acc_sh.at[pl.ds(row0 + off, sz)],
                            out_hbm.at[cid, pl.ds(row0 + off, sz)])

    return k(hk, sctab, tgt, src)


def _make_M(att_l):
    at_w = att_l[:, :UNITS, 0]
    as_w = att_l[:, UNITS:, 0]
    eye = jnp.eye(HEADS, dtype=jnp.float32)
    Mt = jnp.einsum("hu,hk->huk", at_w, eye).reshape(HID, HEADS)
    Ms = jnp.einsum("hu,hk->huk", as_w, eye).reshape(HID, HEADS)
    return jnp.concatenate([Mt, Ms, jnp.zeros((HID, 8), jnp.float32)], axis=1)


def kernel(x, edges, W0, b0, W1, b1, gat_kernels, gat_attn, Wo, bo):
    K0 = gat_kernels[0].transpose(1, 0, 2).reshape(HID, HID)
    K1 = gat_kernels[1].transpose(1, 0, 2).reshape(HID, HID)
    M0 = _make_M(gat_attn[0])
    M1 = _make_M(gat_attn[1])
    S = jnp.repeat(jnp.eye(HEADS, dtype=jnp.float32), UNITS, axis=1)

    xp = jnp.zeros((N_PAD, D), jnp.float32).at[:N].set(x)
    padi = jnp.full((E_PAD - E,), N, jnp.int32)
    tgt = jnp.concatenate([edges[:, 0], padi])
    src = jnp.concatenate([edges[:, 1], padi])

    h0, hk0, st0 = _tc_front(xp, W0, b0.reshape(1, -1), W1, b1.reshape(1, -1),
                             K0, M0)
    acc0 = _sc_gat_layer(hk0, st0, tgt, src)
    h1, hk1, st1 = _tc_mid(acc0[0, :, :HID], acc0[1, :, :HID],
                           acc0[0, :, HID:HID + 4], acc0[1, :, HID:HID + 4],
                           h0, K1, M1, S)
    acc1 = _sc_gat_layer(hk1, st1, tgt, src)
    y = _tc_out(acc1[0, :, :HID], acc1[1, :, :HID],
                acc1[0, :, HID:HID + 4], acc1[1, :, HID:HID + 4],
                h1, Wo, bo.reshape(1, -1), S)
    return y[:N]


# speedup vs baseline: 22.4173x; 1.1306x over previous
import dataclasses
import functools

import jax
import jax.numpy as jnp
from jax import lax
from jax.experimental import pallas as pl
from jax.experimental.pallas import tpu as pltpu
from jax.experimental.pallas import tpu_sc as plsc

N = 10000
E = 320000
D = 128
UNITS = 32
HEADS = 4
HID = UNITS * HEADS
OUT = 2

NC = 2
NS = 16
LANES = 16
NW = NC * NS

N_PAD = 10016
ACCW = 144
CHUNK = 64
E_PAD = 327680
EP_TILE = E_PAD // NW
NCHUNK = EP_TILE // CHUNK
ROWS_PER_TILE = N_PAD // NS

BR = 2504
GRID = N_PAD // BR

_PREC = jax.lax.Precision.HIGHEST


def _dot(a, b):
    return jnp.dot(a, b, precision=_PREC, preferred_element_type=jnp.float32)


def _tc_front(x, W0, b0, W1, b1, K0, M0):
    def body(x_ref, w0_ref, b0_ref, w1_ref, b1_ref, k_ref, m_ref,
             h_ref, hk_ref, st_ref):
        hh = jnp.maximum(_dot(x_ref[...], w0_ref[...]) + b0_ref[...], 0.0)
        hh = jnp.maximum(_dot(hh, w1_ref[...]) + b1_ref[...], 0.0)
        h_ref[...] = hh
        hk = _dot(hh, k_ref[...])
        hk_ref[...] = hk
        st_ref[...] = _dot(hk, m_ref[...])

    full = lambda shape: pl.BlockSpec(shape, lambda i: (0, 0))
    return pl.pallas_call(
        body,
        grid=(GRID,),
        in_specs=[
            pl.BlockSpec((BR, D), lambda i: (i, 0)),
            full((D, HID)), full((1, HID)), full((HID, HID)), full((1, HID)),
            full((HID, HID)), full((HID, 16)),
        ],
        out_specs=[
            pl.BlockSpec((BR, HID), lambda i: (i, 0)),
            pl.BlockSpec((BR, HID), lambda i: (i, 0)),
            pl.BlockSpec((BR, 16), lambda i: (i, 0)),
        ],
        out_shape=[
            jax.ShapeDtypeStruct((N_PAD, HID), jnp.float32),
            jax.ShapeDtypeStruct((N_PAD, HID), jnp.float32),
            jax.ShapeDtypeStruct((N_PAD, 16), jnp.float32),
        ],
    )(x, W0, b0, W1, b1, K0, M0)


def _tc_mid(wA, wB, dA, dB, hprev, K, M, S):
    def body(wa_ref, wb_ref, da_ref, db_ref, hp_ref, k_ref, m_ref, s_ref,
             h_ref, hk_ref, st_ref):
        w = wa_ref[...] + wb_ref[...]
        den = da_ref[...] + db_ref[...]
        den128 = _dot(den, s_ref[...]) + 1e-7
        h1 = jnp.maximum(w / den128, 0.0) + hp_ref[...]
        h_ref[...] = h1
        hk = _dot(h1, k_ref[...])
        hk_ref[...] = hk
        st_ref[...] = _dot(hk, m_ref[...])

    full = lambda shape: pl.BlockSpec(shape, lambda i: (0, 0))
    row = lambda w: pl.BlockSpec((BR, w), lambda i: (i, 0))
    return pl.pallas_call(
        body,
        grid=(GRID,),
        in_specs=[row(HID), row(HID), row(4), row(4), row(HID),
                  full((HID, HID)), full((HID, 16)), full((HEADS, HID))],
        out_specs=[row(HID), row(HID), row(16)],
        out_shape=[
            jax.ShapeDtypeStruct((N_PAD, HID), jnp.float32),
            jax.ShapeDtypeStruct((N_PAD, HID), jnp.float32),
            jax.ShapeDtypeStruct((N_PAD, 16), jnp.float32),
        ],
    )(wA, wB, dA, dB, hprev, K, M, S)


def _tc_out(wA, wB, dA, dB, hprev, Wo, bo, S):
    def body(wa_ref, wb_ref, da_ref, db_ref, hp_ref, wo_ref, bo_ref, s_ref,
             y_ref):
        w = wa_ref[...] + wb_ref[...]
        den = da_ref[...] + db_ref[...]
        den128 = _dot(den, s_ref[...]) + 1e-7
        h2 = jnp.maximum(w / den128, 0.0) + hp_ref[...]
        y_ref[...] = _dot(h2, wo_ref[...]) + bo_ref[...]

    full = lambda shape: pl.BlockSpec(shape, lambda i: (0, 0))
    row = lambda w: pl.BlockSpec((BR, w), lambda i: (i, 0))
    return pl.pallas_call(
        body,
        grid=(GRID,),
        in_specs=[row(HID), row(HID), row(4), row(4), row(HID),
                  full((HID, OUT)), full((1, OUT)), full((HEADS, HID))],
        out_specs=row(OUT),
        out_shape=jax.ShapeDtypeStruct((N_PAD, OUT), jnp.float32),
    )(wA, wB, dA, dB, hprev, Wo, bo, S)


def _sc_gat_layer(hk, sctab, tgt, src):
    mesh = plsc.VectorSubcoreMesh(core_axis_name="c", subcore_axis_name="s")
    cp = pltpu.CompilerParams()
    if "needs_layout_passes" in pltpu.CompilerParams.__dataclass_fields__:
        cp = dataclasses.replace(cp, needs_layout_passes=False)
    if "use_tc_tiling_on_sc" in pltpu.CompilerParams.__dataclass_fields__:
        cp = dataclasses.replace(cp, use_tc_tiling_on_sc=False)

    scratch_types=(
        [pltpu.VMEM((CHUNK,), jnp.int32) for _ in range(4)]
        + [pltpu.VMEM((CHUNK,), jnp.int32) for _ in range(4)]
        + [pltpu.VMEM((CHUNK, 16), jnp.float32) for _ in range(2)]
        + [pltpu.VMEM((CHUNK, 16), jnp.float32) for _ in range(2)]
        + [pltpu.VMEM((CHUNK, HID), jnp.float32) for _ in range(2)]
        + [pltpu.VMEM((CHUNK, ACCW), jnp.float32) for _ in range(2)]
        + [pltpu.VMEM_SHARED((N_PAD, ACCW), jnp.float32)]
        + [pltpu.SemaphoreType.DMA for _ in range(8)]
    )

    @functools.partial(
        pl.kernel,
        out_type=jax.ShapeDtypeStruct((NC, N_PAD, ACCW), jnp.float32),
        mesh=mesh,
        compiler_params=cp,
        scratch_types=scratch_types,
    )
    def k(hk_hbm, sctab_hbm, tgt_hbm, src_hbm, out_hbm,
          t0, t1, t2, t3, s0, s1, s2, s3, a0, a1, b0, b1, r0, r1, o0, o1,
          acc_sh, i_sem0, i_sem1, i_sem2, i_sem3, gsem0, gsem1, ssem0, ssem1):
        cid = lax.axis_index("c")
        sid = lax.axis_index("s")
        wid = cid * NS + sid
        tbuf, sbuf = (t0, t1, t2, t3), (s0, s1, s2, s3)
        abuf, bbuf, rbuf, obuf = (a0, a1), (b0, b1), (r0, r1), (o0, o1)
        isem = (i_sem0, i_sem1, i_sem2, i_sem3)
        gsem, ssem = (gsem0, gsem1), (ssem0, ssem1)

        zero16 = jnp.zeros((LANES,), jnp.float32)

        for o in obuf:
            @pl.loop(0, CHUNK)
            def _zero_rows(i):
                @pl.loop(0, ACCW, step=LANES)
                def _zero_cols(j):
                    o[i, pl.ds(j, LANES)] = zero16

        row0 = sid * ROWS_PER_TILE
        for off in range(0, ROWS_PER_TILE, CHUNK):
            sz = min(CHUNK, ROWS_PER_TILE - off)
            pltpu.sync_copy(o0.at[pl.ds(0, sz)],
                            acc_sh.at[pl.ds(row0 + off, sz)])
        plsc.subcore_barrier()

        iota = lax.iota(jnp.int32, LANES)
        idiv4 = lax.shift_right_logical(iota, 2)
        imod4 = iota & 3
        col_a = imod4
        col_b = imod4 + HEADS
        col_p = imod4 + HID
        midx = [jnp.full((LANES,), j, jnp.int32) for j in range(LANES)]

        ebase = wid * EP_TILE

        def start_idx(ci, ib):
            base = ebase + ci * CHUNK
            pltpu.async_copy(tgt_hbm.at[pl.ds(base, CHUNK)], tbuf[ib], isem[ib])
            pltpu.async_copy(src_hbm.at[pl.ds(base, CHUNK)], sbuf[ib], isem[ib])

        def wait_idx(ci, ib):
            base = ebase + ci * CHUNK
            pltpu.make_async_copy(tgt_hbm.at[pl.ds(base, CHUNK)], tbuf[ib],
                                  isem[ib]).wait()
            pltpu.make_async_copy(src_hbm.at[pl.ds(base, CHUNK)], sbuf[ib],
                                  isem[ib]).wait()

        def start_gathers(gb, ib):
            pltpu.async_copy(sctab_hbm.at[tbuf[ib]], abuf[gb], gsem[gb])
            pltpu.async_copy(sctab_hbm.at[sbuf[ib]], bbuf[gb], gsem[gb])
            pltpu.async_copy(hk_hbm.at[sbuf[ib]], rbuf[gb], gsem[gb])

        def wait_gathers(gb, ib):
            pltpu.make_async_copy(sctab_hbm.at[tbuf[ib]], abuf[gb],
                                  gsem[gb]).wait()
            pltpu.make_async_copy(sctab_hbm.at[sbuf[ib]], bbuf[gb],
                                  gsem[gb]).wait()
            pltpu.make_async_copy(hk_hbm.at[sbuf[ib]], rbuf[gb],
                                  gsem[gb]).wait()

        def wait_scatter(gb, ib):
            pltpu.make_async_copy(obuf[gb], acc_sh.at[tbuf[ib]],
                                  ssem[gb]).wait()

        def compute_chunk(a_v, b_v, rows_v, orow_v):
            @plsc.parallel_loop(0, CHUNK // 4, unroll=2)
            def _group(g):
                e0 = g * 4
                rowidx = idiv4 + e0
                va = plsc.load_gather(a_v, [rowidx, col_a])
                vb = plsc.load_gather(b_v, [rowidx, col_b])
                s = va + vb
                s = jnp.maximum(s, 0.2 * s)
                s = jnp.clip(s, -2.0, 2.0)
                p4 = jnp.exp(s)
                plsc.store_scatter(orow_v, [rowidx, col_p], p4)
                ms = [jnp.take_along_axis(p4, midx[j], axis=0,
                                          mode="promise_in_bounds")
                      for j in range(16)]
                for kk in range(4):
                    e = e0 + kk
                    loads = [rows_v[e, pl.ds(LANES * c, LANES)]
                             for c in range(8)]
                    prods = [loads[c] * ms[4 * kk + c // 2] for c in range(8)]
                    for c in range(8):
                        orow_v[e, pl.ds(LANES * c, LANES)] = prods[c]

        start_idx(0, 0)
        start_idx(1, 1)
        wait_idx(0, 0)
        start_gathers(0, 0)

        @pl.loop(0, NCHUNK, step=4)
        def _quad(ci):
            for b in range(4):
                cur = ci + b
                gb = b % 2

                @pl.when(cur >= 2)
                def _ws():
                    wait_scatter(gb, (b - 2) % 4)

                @pl.when(cur + 2 < NCHUNK)
                def _pi():
                    start_idx(cur + 2, (b + 2) % 4)

                @pl.when(cur + 1 < NCHUNK)
                def _pg():
                    wait_idx(cur + 1, (b + 1) % 4)
                    start_gathers(1 - gb, (b + 1) % 4)

                wait_gathers(gb, b)
                compute_chunk(abuf[gb], bbuf[gb], rbuf[gb], obuf[gb])
                pltpu.async_copy(obuf[gb], acc_sh.at[tbuf[b]], ssem[gb],
                                 add=True)

        wait_scatter(0, 2)
        wait_scatter(1, 3)
        plsc.subcore_barrier()
        for off in range(0, ROWS_PER_TILE, CHUNK):
            sz = min(CHUNK, ROWS_PER_TILE - off)
            pltpu.sync_copy(acc_sh.at[pl.ds(row0 + off, sz)],
                            out_hbm.at[cid, pl.ds(row0 + off, sz)])

    return k(hk, sctab, tgt, src)


def _make_M(att_l):
    at_w = att_l[:, :UNITS, 0]
    as_w = att_l[:, UNITS:, 0]
    eye = jnp.eye(HEADS, dtype=jnp.float32)
    Mt = jnp.einsum("hu,hk->huk", at_w, eye).reshape(HID, HEADS)
    Ms = jnp.einsum("hu,hk->huk", as_w, eye).reshape(HID, HEADS)
    return jnp.concatenate([Mt, Ms, jnp.zeros((HID, 8), jnp.float32)], axis=1)


def kernel(x, edges, W0, b0, W1, b1, gat_kernels, gat_attn, Wo, bo):
    K0 = gat_kernels[0].transpose(1, 0, 2).reshape(HID, HID)
    K1 = gat_kernels[1].transpose(1, 0, 2).reshape(HID, HID)
    M0 = _make_M(gat_attn[0])
    M1 = _make_M(gat_attn[1])
    S = jnp.repeat(jnp.eye(HEADS, dtype=jnp.float32), UNITS, axis=1)

    xp = jnp.zeros((N_PAD, D), jnp.float32).at[:N].set(x)
    padi = jnp.full((E_PAD - E,), N, jnp.int32)
    tgt = jnp.concatenate([edges[:, 0], padi])
    src = jnp.concatenate([edges[:, 1], padi])

    h0, hk0, st0 = _tc_front(xp, W0, b0.reshape(1, -1), W1, b1.reshape(1, -1),
                             K0, M0)
    acc0 = _sc_gat_layer(hk0, st0, tgt, src)
    h1, hk1, st1 = _tc_mid(acc0[0, :, :HID], acc0[1, :, :HID],
                           acc0[0, :, HID:HID + 4], acc0[1, :, HID:HID + 4],
                           h0, K1, M1, S)
    acc1 = _sc_gat_layer(hk1, st1, tgt, src)
    y = _tc_out(acc1[0, :, :HID], acc1[1, :, :HID],
                acc1[0, :, HID:HID + 4], acc1[1, :, HID:HID + 4],
                h1, Wo, bo.reshape(1, -1), S)
    return y[:N]


# speedup vs baseline: 40.6785x; 1.8146x over previous
import dataclasses
import functools

import jax
import jax.numpy as jnp
from jax import lax
from jax.experimental import pallas as pl
from jax.experimental.pallas import tpu as pltpu
from jax.experimental.pallas import tpu_sc as plsc

N = 10000
E = 320000
D = 128
UNITS = 32
HEADS = 4
HID = UNITS * HEADS
OUT = 2

NC = 2
NS = 16
LANES = 16
NW = NC * NS

N_PAD = 10016
ACCW = 144
CHUNK = 64
E_PAD = 327680
EP_TILE = E_PAD // NW
NCHUNK = EP_TILE // CHUNK
ROWS_PER_TILE = N_PAD // NS

BR = 2504
GRID = N_PAD // BR

_PREC = jax.lax.Precision.HIGHEST


def _dot(a, b):
    return jnp.dot(a, b, precision=_PREC, preferred_element_type=jnp.float32)


def _tc_front(x, W0, b0, W1, b1, K0, M0):
    def body(x_ref, w0_ref, b0_ref, w1_ref, b1_ref, k_ref, m_ref,
             h_ref, hk_ref, st_ref):
        hh = jnp.maximum(_dot(x_ref[...], w0_ref[...]) + b0_ref[...], 0.0)
        hh = jnp.maximum(_dot(hh, w1_ref[...]) + b1_ref[...], 0.0)
        h_ref[...] = hh
        hk = _dot(hh, k_ref[...])
        hk_ref[...] = hk
        st_ref[...] = _dot(hk, m_ref[...])

    full = lambda shape: pl.BlockSpec(shape, lambda i: (0, 0))
    return pl.pallas_call(
        body,
        grid=(GRID,),
        in_specs=[
            pl.BlockSpec((BR, D), lambda i: (i, 0)),
            full((D, HID)), full((1, HID)), full((HID, HID)), full((1, HID)),
            full((HID, HID)), full((HID, 16)),
        ],
        out_specs=[
            pl.BlockSpec((BR, HID), lambda i: (i, 0)),
            pl.BlockSpec((BR, HID), lambda i: (i, 0)),
            pl.BlockSpec((BR, 16), lambda i: (i, 0)),
        ],
        out_shape=[
            jax.ShapeDtypeStruct((N_PAD, HID), jnp.float32),
            jax.ShapeDtypeStruct((N_PAD, HID), jnp.float32),
            jax.ShapeDtypeStruct((N_PAD, 16), jnp.float32),
        ],
    )(x, W0, b0, W1, b1, K0, M0)


def _tc_mid(wA, wB, dA, dB, hprev, K, M, S):
    def body(wa_ref, wb_ref, da_ref, db_ref, hp_ref, k_ref, m_ref, s_ref,
             h_ref, hk_ref, st_ref):
        w = wa_ref[...] + wb_ref[...]
        den = da_ref[...] + db_ref[...]
        den128 = _dot(den, s_ref[...]) + 1e-7
        h1 = jnp.maximum(w / den128, 0.0) + hp_ref[...]
        h_ref[...] = h1
        hk = _dot(h1, k_ref[...])
        hk_ref[...] = hk
        st_ref[...] = _dot(hk, m_ref[...])

    full = lambda shape: pl.BlockSpec(shape, lambda i: (0, 0))
    row = lambda w: pl.BlockSpec((BR, w), lambda i: (i, 0))
    return pl.pallas_call(
        body,
        grid=(GRID,),
        in_specs=[row(HID), row(HID), row(4), row(4), row(HID),
                  full((HID, HID)), full((HID, 16)), full((HEADS, HID))],
        out_specs=[row(HID), row(HID), row(16)],
        out_shape=[
            jax.ShapeDtypeStruct((N_PAD, HID), jnp.float32),
            jax.ShapeDtypeStruct((N_PAD, HID), jnp.float32),
            jax.ShapeDtypeStruct((N_PAD, 16), jnp.float32),
        ],
    )(wA, wB, dA, dB, hprev, K, M, S)


def _tc_out(wA, wB, dA, dB, hprev, Wo, bo, S):
    def body(wa_ref, wb_ref, da_ref, db_ref, hp_ref, wo_ref, bo_ref, s_ref,
             y_ref):
        w = wa_ref[...] + wb_ref[...]
        den = da_ref[...] + db_ref[...]
        den128 = _dot(den, s_ref[...]) + 1e-7
        h2 = jnp.maximum(w / den128, 0.0) + hp_ref[...]
        y_ref[...] = _dot(h2, wo_ref[...]) + bo_ref[...]

    full = lambda shape: pl.BlockSpec(shape, lambda i: (0, 0))
    row = lambda w: pl.BlockSpec((BR, w), lambda i: (i, 0))
    return pl.pallas_call(
        body,
        grid=(GRID,),
        in_specs=[row(HID), row(HID), row(4), row(4), row(HID),
                  full((HID, OUT)), full((1, OUT)), full((HEADS, HID))],
        out_specs=row(OUT),
        out_shape=jax.ShapeDtypeStruct((N_PAD, OUT), jnp.float32),
    )(wA, wB, dA, dB, hprev, Wo, bo, S)


def _sc_gat_layer(hk, sctab, tgt, src):
    mesh = plsc.VectorSubcoreMesh(core_axis_name="c", subcore_axis_name="s")
    cp = pltpu.CompilerParams()
    if "needs_layout_passes" in pltpu.CompilerParams.__dataclass_fields__:
        cp = dataclasses.replace(cp, needs_layout_passes=False)
    if "use_tc_tiling_on_sc" in pltpu.CompilerParams.__dataclass_fields__:
        cp = dataclasses.replace(cp, use_tc_tiling_on_sc=False)

    scratch_types=(
        [pltpu.VMEM((CHUNK,), jnp.int32) for _ in range(4)]
        + [pltpu.VMEM((CHUNK,), jnp.int32) for _ in range(4)]
        + [pltpu.VMEM((CHUNK, 16), jnp.float32) for _ in range(2)]
        + [pltpu.VMEM((CHUNK, 16), jnp.float32) for _ in range(2)]
        + [pltpu.VMEM((CHUNK, HID), jnp.float32) for _ in range(2)]
        + [pltpu.VMEM((CHUNK, ACCW), jnp.float32) for _ in range(2)]
        + [pltpu.VMEM_SHARED((N_PAD, ACCW), jnp.float32)]
        + [pltpu.SemaphoreType.DMA for _ in range(8)]
    )

    @functools.partial(
        pl.kernel,
        out_type=jax.ShapeDtypeStruct((NC, N_PAD, ACCW), jnp.float32),
        mesh=mesh,
        compiler_params=cp,
        scratch_types=scratch_types,
    )
    def k(hk_hbm, sctab_hbm, tgt_hbm, src_hbm, out_hbm,
          t0, t1, t2, t3, s0, s1, s2, s3, a0, a1, b0, b1, r0, r1, o0, o1,
          acc_sh, i_sem0, i_sem1, i_sem2, i_sem3, gsem0, gsem1, ssem0, ssem1):
        cid = lax.axis_index("c")
        sid = lax.axis_index("s")
        wid = cid * NS + sid
        tbuf, sbuf = (t0, t1, t2, t3), (s0, s1, s2, s3)
        abuf, bbuf, rbuf, obuf = (a0, a1), (b0, b1), (r0, r1), (o0, o1)
        isem = (i_sem0, i_sem1, i_sem2, i_sem3)
        gsem, ssem = (gsem0, gsem1), (ssem0, ssem1)

        zero16 = jnp.zeros((LANES,), jnp.float32)

        for o in obuf:
            @pl.loop(0, CHUNK)
            def _zero_rows(i):
                @pl.loop(0, ACCW, step=LANES)
                def _zero_cols(j):
                    o[i, pl.ds(j, LANES)] = zero16

        row0 = sid * ROWS_PER_TILE
        for off in range(0, ROWS_PER_TILE, CHUNK):
            sz = min(CHUNK, ROWS_PER_TILE - off)
            pltpu.sync_copy(o0.at[pl.ds(0, sz)],
                            acc_sh.at[pl.ds(row0 + off, sz)])
        plsc.subcore_barrier()

        iota = lax.iota(jnp.int32, LANES)
        idiv4 = lax.shift_right_logical(iota, 2)
        imod4 = iota & 3
        col_a = imod4
        col_b = imod4 + HEADS
        col_p = imod4 + HID
        midx = [jnp.full((LANES,), j, jnp.int32) for j in range(LANES)]

        ebase = wid * EP_TILE

        def start_idx(ci, ib):
            base = ebase + ci * CHUNK
            pltpu.async_copy(tgt_hbm.at[pl.ds(base, CHUNK)], tbuf[ib], isem[ib])
            pltpu.async_copy(src_hbm.at[pl.ds(base, CHUNK)], sbuf[ib], isem[ib])

        def wait_idx(ci, ib):
            base = ebase + ci * CHUNK
            pltpu.make_async_copy(tgt_hbm.at[pl.ds(base, CHUNK)], tbuf[ib],
                                  isem[ib]).wait()
            pltpu.make_async_copy(src_hbm.at[pl.ds(base, CHUNK)], sbuf[ib],
                                  isem[ib]).wait()

        def start_gathers(gb, ib):
            pltpu.async_copy(sctab_hbm.at[tbuf[ib]], abuf[gb], gsem[gb])
            pltpu.async_copy(sctab_hbm.at[sbuf[ib]], bbuf[gb], gsem[gb])
            pltpu.async_copy(hk_hbm.at[sbuf[ib]], rbuf[gb], gsem[gb])

        def wait_gathers(gb, ib):
            pltpu.make_async_copy(sctab_hbm.at[tbuf[ib]], abuf[gb],
                                  gsem[gb]).wait()
            pltpu.make_async_copy(sctab_hbm.at[sbuf[ib]], bbuf[gb],
                                  gsem[gb]).wait()
            pltpu.make_async_copy(hk_hbm.at[sbuf[ib]], rbuf[gb],
                                  gsem[gb]).wait()

        def wait_scatter(gb, ib):
            pltpu.make_async_copy(obuf[gb], acc_sh.at[tbuf[ib]],
                                  ssem[gb]).wait()

        def compute_chunk(a_v, b_v, rows_v, orow_v):
            @plsc.parallel_loop(0, CHUNK // 4, unroll=2)
            def _group(g):
                e0 = g * 4
                rowidx = idiv4 + e0
                va = plsc.load_gather(a_v, [rowidx, col_a])
                vb = plsc.load_gather(b_v, [rowidx, col_b])
                s = va + vb
                s = jnp.maximum(s, 0.2 * s)
                s = jnp.clip(s, -2.0, 2.0)
                p4 = jnp.exp(s)
                plsc.store_scatter(orow_v, [rowidx, col_p], p4)
                ms = [jnp.take_along_axis(p4, midx[j], axis=0,
                                          mode="promise_in_bounds")
                      for j in range(16)]
                for kk in range(4):
                    e = e0 + kk
                    loads = [rows_v[e, pl.ds(LANES * c, LANES)]
                             for c in range(8)]
                    prods = [loads[c] * ms[4 * kk + c // 2] for c in range(8)]
                    for c in range(8):
                        orow_v[e, pl.ds(LANES * c, LANES)] = prods[c]

        start_idx(0, 0)
        start_idx(1, 1)
        wait_idx(0, 0)
        start_gathers(0, 0)

        @pl.loop(0, NCHUNK, step=4)
        def _quad(ci):
            for b in range(4):
                cur = ci + b
                gb = b % 2

                @pl.when(cur >= 2)
                def _ws():
                    wait_scatter(gb, (b - 2) % 4)

                @pl.when(cur + 2 < NCHUNK)
                def _pi():
                    start_idx(cur + 2, (b + 2) % 4)

                @pl.when(cur + 1 < NCHUNK)
                def _pg():
                    wait_idx(cur + 1, (b + 1) % 4)
                    start_gathers(1 - gb, (b + 1) % 4)

                wait_gathers(gb, b)
                compute_chunk(abuf[gb], bbuf[gb], rbuf[gb], obuf[gb])
                pltpu.async_copy(obuf[gb], acc_sh.at[tbuf[b]], ssem[gb],
                                 add=True)

        wait_scatter(0, 2)
        wait_scatter(1, 3)
        plsc.subcore_barrier()
        for off in range(0, ROWS_PER_TILE, CHUNK):
            sz = min(CHUNK, ROWS_PER_TILE - off)
            pltpu.sync_copy(acc_sh.at[pl.ds(row0 + off, sz)],
                            out_hbm.at[cid, pl.ds(row0 + off, sz)])

    return k(hk, sctab, tgt, src)


def _make_M(att_l):
    at_w = att_l[:, :UNITS, 0]
    as_w = att_l[:, UNITS:, 0]
    eye = jnp.eye(HEADS, dtype=jnp.float32)
    Mt = jnp.einsum("hu,hk->huk", at_w, eye).reshape(HID, HEADS)
    Ms = jnp.einsum("hu,hk->huk", as_w, eye).reshape(HID, HEADS)
    return jnp.concatenate([Mt, Ms, jnp.zeros((HID, 8), jnp.float32)], axis=1)


def kernel(x, edges, W0, b0, W1, b1, gat_kernels, gat_attn, Wo, bo):
    K0 = gat_kernels[0].transpose(1, 0, 2).reshape(HID, HID)
    K1 = gat_kernels[1].transpose(1, 0, 2).reshape(HID, HID)
    M0 = _make_M(gat_attn[0])
    M1 = _make_M(gat_attn[1])
    S = jnp.repeat(jnp.eye(HEADS, dtype=jnp.float32), UNITS, axis=1)

    xp = jnp.zeros((N_PAD, D), jnp.float32).at[:N].set(x)
    padi = N + (jnp.arange(E_PAD - E, dtype=jnp.int32) % (N_PAD - N))
    interleave = lambda v: v.reshape(EP_TILE, NW).T.reshape(-1)
    tgt = interleave(jnp.concatenate([edges[:, 0], padi]))
    src = interleave(jnp.concatenate([edges[:, 1], padi]))

    h0, hk0, st0 = _tc_front(xp, W0, b0.reshape(1, -1), W1, b1.reshape(1, -1),
                             K0, M0)
    acc0 = _sc_gat_layer(hk0, st0, tgt, src)
    h1, hk1, st1 = _tc_mid(acc0[0, :, :HID], acc0[1, :, :HID],
                           acc0[0, :, HID:HID + 4], acc0[1, :, HID:HID + 4],
                           h0, K1, M1, S)
    acc1 = _sc_gat_layer(hk1, st1, tgt, src)
    y = _tc_out(acc1[0, :, :HID], acc1[1, :, :HID],
                acc1[0, :, HID:HID + 4], acc1[1, :, HID:HID + 4],
                h1, Wo, bo.reshape(1, -1), S)
    return y[:N]


# speedup vs baseline: 50.0115x; 1.2294x over previous
import dataclasses
import functools

import jax
import jax.numpy as jnp
from jax import lax
from jax.experimental import pallas as pl
from jax.experimental.pallas import tpu as pltpu
from jax.experimental.pallas import tpu_sc as plsc

N = 10000
E = 320000
D = 128
UNITS = 32
HEADS = 4
HID = UNITS * HEADS
OUT = 2

NC = 2
NS = 16
LANES = 16
NW = NC * NS

N_PAD = 10016
ACCW = 144
CHUNK = 64
E_PAD = 327680
EP_TILE = E_PAD // NW
NCHUNK = EP_TILE // CHUNK
ROWS_PER_TILE = N_PAD // NS

BR = 2504
GRID = N_PAD // BR

_PREC = jax.lax.Precision.DEFAULT


def _dot(a, b):
    return jnp.dot(a, b, precision=_PREC, preferred_element_type=jnp.float32)


def _tc_front(x, W0, b0, W1, b1, K0, M0):
    def body(x_ref, w0_ref, b0_ref, w1_ref, b1_ref, k_ref, m_ref,
             h_ref, hk_ref, st_ref):
        hh = jnp.maximum(_dot(x_ref[...], w0_ref[...]) + b0_ref[...], 0.0)
        hh = jnp.maximum(_dot(hh, w1_ref[...]) + b1_ref[...], 0.0)
        h_ref[...] = hh
        hk = _dot(hh, k_ref[...])
        hk_ref[...] = hk
        st_ref[...] = _dot(hk, m_ref[...])

    full = lambda shape: pl.BlockSpec(shape, lambda i: (0, 0))
    return pl.pallas_call(
        body,
        grid=(GRID,),
        in_specs=[
            pl.BlockSpec((BR, D), lambda i: (i, 0)),
            full((D, HID)), full((1, HID)), full((HID, HID)), full((1, HID)),
            full((HID, HID)), full((HID, 16)),
        ],
        out_specs=[
            pl.BlockSpec((BR, HID), lambda i: (i, 0)),
            pl.BlockSpec((BR, HID), lambda i: (i, 0)),
            pl.BlockSpec((BR, 16), lambda i: (i, 0)),
        ],
        out_shape=[
            jax.ShapeDtypeStruct((N_PAD, HID), jnp.float32),
            jax.ShapeDtypeStruct((N_PAD, HID), jnp.float32),
            jax.ShapeDtypeStruct((N_PAD, 16), jnp.float32),
        ],
    )(x, W0, b0, W1, b1, K0, M0)


def _tc_mid(wA, wB, dA, dB, hprev, K, M, S):
    def body(wa_ref, wb_ref, da_ref, db_ref, hp_ref, k_ref, m_ref, s_ref,
             h_ref, hk_ref, st_ref):
        w = wa_ref[...] + wb_ref[...]
        den = da_ref[...] + db_ref[...]
        den128 = _dot(den, s_ref[...]) + 1e-7
        h1 = jnp.maximum(w / den128, 0.0) + hp_ref[...]
        h_ref[...] = h1
        hk = _dot(h1, k_ref[...])
        hk_ref[...] = hk
        st_ref[...] = _dot(hk, m_ref[...])

    full = lambda shape: pl.BlockSpec(shape, lambda i: (0, 0))
    row = lambda w: pl.BlockSpec((BR, w), lambda i: (i, 0))
    return pl.pallas_call(
        body,
        grid=(GRID,),
        in_specs=[row(HID), row(HID), row(16), row(16), row(HID),
                  full((HID, HID)), full((HID, 16)), full((16, HID))],
        out_specs=[row(HID), row(HID), row(16)],
        out_shape=[
            jax.ShapeDtypeStruct((N_PAD, HID), jnp.float32),
            jax.ShapeDtypeStruct((N_PAD, HID), jnp.float32),
            jax.ShapeDtypeStruct((N_PAD, 16), jnp.float32),
        ],
    )(wA, wB, dA, dB, hprev, K, M, S)


def _tc_out(wA, wB, dA, dB, hprev, Wo, bo, S):
    def body(wa_ref, wb_ref, da_ref, db_ref, hp_ref, wo_ref, bo_ref, s_ref,
             y_ref):
        w = wa_ref[...] + wb_ref[...]
        den = da_ref[...] + db_ref[...]
        den128 = _dot(den, s_ref[...]) + 1e-7
        h2 = jnp.maximum(w / den128, 0.0) + hp_ref[...]
        y_ref[...] = _dot(h2, wo_ref[...]) + bo_ref[...]

    full = lambda shape: pl.BlockSpec(shape, lambda i: (0, 0))
    row = lambda w: pl.BlockSpec((BR, w), lambda i: (i, 0))
    return pl.pallas_call(
        body,
        grid=(GRID,),
        in_specs=[row(HID), row(HID), row(16), row(16), row(HID),
                  full((HID, OUT)), full((1, OUT)), full((16, HID))],
        out_specs=row(OUT),
        out_shape=jax.ShapeDtypeStruct((N_PAD, OUT), jnp.float32),
    )(wA, wB, dA, dB, hprev, Wo, bo, S)


def _sc_gat_layer(hk, sctab, tgt, src):
    mesh = plsc.VectorSubcoreMesh(core_axis_name="c", subcore_axis_name="s")
    cp = pltpu.CompilerParams()
    if "needs_layout_passes" in pltpu.CompilerParams.__dataclass_fields__:
        cp = dataclasses.replace(cp, needs_layout_passes=False)
    if "use_tc_tiling_on_sc" in pltpu.CompilerParams.__dataclass_fields__:
        cp = dataclasses.replace(cp, use_tc_tiling_on_sc=False)

    scratch_types=(
        [pltpu.VMEM((CHUNK,), jnp.int32) for _ in range(4)]
        + [pltpu.VMEM((CHUNK,), jnp.int32) for _ in range(4)]
        + [pltpu.VMEM((CHUNK, 16), jnp.float32) for _ in range(2)]
        + [pltpu.VMEM((CHUNK, 16), jnp.float32) for _ in range(2)]
        + [pltpu.VMEM((CHUNK, HID), jnp.float32) for _ in range(2)]
        + [pltpu.VMEM((CHUNK, ACCW), jnp.float32) for _ in range(2)]
        + [pltpu.VMEM_SHARED((N_PAD, ACCW), jnp.float32)]
        + [pltpu.SemaphoreType.DMA for _ in range(8)]
    )

    @functools.partial(
        pl.kernel,
        out_type=[jax.ShapeDtypeStruct((NC, N_PAD, HID), jnp.float32),
                  jax.ShapeDtypeStruct((NC, N_PAD, 16), jnp.float32)],
        mesh=mesh,
        compiler_params=cp,
        scratch_types=scratch_types,
    )
    def k(hk_hbm, sctab_hbm, tgt_hbm, src_hbm, wsum_hbm, den_hbm,
          t0, t1, t2, t3, s0, s1, s2, s3, a0, a1, b0, b1, r0, r1, o0, o1,
          acc_sh, i_sem0, i_sem1, i_sem2, i_sem3, gsem0, gsem1, ssem0, ssem1):
        cid = lax.axis_index("c")
        sid = lax.axis_index("s")
        wid = cid * NS + sid
        tbuf, sbuf = (t0, t1, t2, t3), (s0, s1, s2, s3)
        abuf, bbuf, rbuf, obuf = (a0, a1), (b0, b1), (r0, r1), (o0, o1)
        isem = (i_sem0, i_sem1, i_sem2, i_sem3)
        gsem, ssem = (gsem0, gsem1), (ssem0, ssem1)

        zero16 = jnp.zeros((LANES,), jnp.float32)

        for o in obuf:
            @pl.loop(0, CHUNK)
            def _zero_rows(i):
                @pl.loop(0, ACCW, step=LANES)
                def _zero_cols(j):
                    o[i, pl.ds(j, LANES)] = zero16

        row0 = sid * ROWS_PER_TILE
        for off in range(0, ROWS_PER_TILE, CHUNK):
            sz = min(CHUNK, ROWS_PER_TILE - off)
            pltpu.sync_copy(o0.at[pl.ds(0, sz)],
                            acc_sh.at[pl.ds(row0 + off, sz)])
        plsc.subcore_barrier()

        iota = lax.iota(jnp.int32, LANES)
        idiv4 = lax.shift_right_logical(iota, 2)
        imod4 = iota & 3
        col_a = imod4
        col_b = imod4 + HEADS
        col_p = imod4 + HID
        midx = [jnp.full((LANES,), j, jnp.int32) for j in range(LANES)]

        ebase = wid * EP_TILE

        def start_idx(ci, ib):
            base = ebase + ci * CHUNK
            pltpu.async_copy(tgt_hbm.at[pl.ds(base, CHUNK)], tbuf[ib], isem[ib])
            pltpu.async_copy(src_hbm.at[pl.ds(base, CHUNK)], sbuf[ib], isem[ib])

        def wait_idx(ci, ib):
            base = ebase + ci * CHUNK
            pltpu.make_async_copy(tgt_hbm.at[pl.ds(base, CHUNK)], tbuf[ib],
                                  isem[ib]).wait()
            pltpu.make_async_copy(src_hbm.at[pl.ds(base, CHUNK)], sbuf[ib],
                                  isem[ib]).wait()

        def start_gathers(gb, ib):
            pltpu.async_copy(sctab_hbm.at[tbuf[ib]], abuf[gb], gsem[gb])
            pltpu.async_copy(sctab_hbm.at[sbuf[ib]], bbuf[gb], gsem[gb])
            pltpu.async_copy(hk_hbm.at[sbuf[ib]], rbuf[gb], gsem[gb])

        def wait_gathers(gb, ib):
            pltpu.make_async_copy(sctab_hbm.at[tbuf[ib]], abuf[gb],
                                  gsem[gb]).wait()
            pltpu.make_async_copy(sctab_hbm.at[sbuf[ib]], bbuf[gb],
                                  gsem[gb]).wait()
            pltpu.make_async_copy(hk_hbm.at[sbuf[ib]], rbuf[gb],
                                  gsem[gb]).wait()

        def wait_scatter(gb, ib):
            pltpu.make_async_copy(obuf[gb], acc_sh.at[tbuf[ib]],
                                  ssem[gb]).wait()

        def compute_chunk(a_v, b_v, rows_v, orow_v):
            @plsc.parallel_loop(0, CHUNK // 4, unroll=2)
            def _group(g):
                e0 = g * 4
                rowidx = idiv4 + e0
                va = plsc.load_gather(a_v, [rowidx, col_a])
                vb = plsc.load_gather(b_v, [rowidx, col_b])
                s = va + vb
                s = jnp.maximum(s, 0.2 * s)
                s = jnp.clip(s, -2.0, 2.0)
                p4 = jnp.exp(s)
                plsc.store_scatter(orow_v, [rowidx, col_p], p4)
                ms = [jnp.take_along_axis(p4, midx[j], axis=0,
                                          mode="promise_in_bounds")
                      for j in range(16)]
                for kk in range(4):
                    e = e0 + kk
                    loads = [rows_v[e, pl.ds(LANES * c, LANES)]
                             for c in range(8)]
                    prods = [loads[c] * ms[4 * kk + c // 2] for c in range(8)]
                    for c in range(8):
                        orow_v[e, pl.ds(LANES * c, LANES)] = prods[c]

        start_idx(0, 0)
        start_idx(1, 1)
        wait_idx(0, 0)
        start_gathers(0, 0)

        @pl.loop(0, NCHUNK, step=4)
        def _quad(ci):
            for b in range(4):
                cur = ci + b
                gb = b % 2

                @pl.when(cur >= 2)
                def _ws():
                    wait_scatter(gb, (b - 2) % 4)

                @pl.when(cur + 2 < NCHUNK)
                def _pi():
                    start_idx(cur + 2, (b + 2) % 4)

                @pl.when(cur + 1 < NCHUNK)
                def _pg():
                    wait_idx(cur + 1, (b + 1) % 4)
                    start_gathers(1 - gb, (b + 1) % 4)

                wait_gathers(gb, b)
                compute_chunk(abuf[gb], bbuf[gb], rbuf[gb], obuf[gb])
                pltpu.async_copy(obuf[gb], acc_sh.at[tbuf[b]], ssem[gb],
                                 add=True)

        wait_scatter(0, 2)
        wait_scatter(1, 3)
        plsc.subcore_barrier()
        for off in range(0, ROWS_PER_TILE, CHUNK):
            sz = min(CHUNK, ROWS_PER_TILE - off)
            pltpu.sync_copy(
                acc_sh.at[pl.ds(row0 + off, sz), pl.ds(0, HID)],
                wsum_hbm.at[cid, pl.ds(row0 + off, sz)])
            pltpu.sync_copy(
                acc_sh.at[pl.ds(row0 + off, sz), pl.ds(HID, 16)],
                den_hbm.at[cid, pl.ds(row0 + off, sz)])

    return k(hk, sctab, tgt, src)


def _make_M(att_l):
    at_w = att_l[:, :UNITS, 0]
    as_w = att_l[:, UNITS:, 0]
    eye = jnp.eye(HEADS, dtype=jnp.float32)
    Mt = jnp.einsum("hu,hk->huk", at_w, eye).reshape(HID, HEADS)
    Ms = jnp.einsum("hu,hk->huk", as_w, eye).reshape(HID, HEADS)
    return jnp.concatenate([Mt, Ms, jnp.zeros((HID, 8), jnp.float32)], axis=1)


def kernel(x, edges, W0, b0, W1, b1, gat_kernels, gat_attn, Wo, bo):
    K0 = gat_kernels[0].transpose(1, 0, 2).reshape(HID, HID)
    K1 = gat_kernels[1].transpose(1, 0, 2).reshape(HID, HID)
    M0 = _make_M(gat_attn[0])
    M1 = _make_M(gat_attn[1])
    S = jnp.concatenate(
        [jnp.repeat(jnp.eye(HEADS, dtype=jnp.float32), UNITS, axis=1),
         jnp.zeros((12, HID), jnp.float32)], axis=0)

    xp = jnp.zeros((N_PAD, D), jnp.float32).at[:N].set(x)
    padi = N + (jnp.arange(E_PAD - E, dtype=jnp.int32) % (N_PAD - N))
    interleave = lambda v: v.reshape(EP_TILE, NW).T.reshape(-1)
    tgt = interleave(jnp.concatenate([edges[:, 0], padi]))
    src = interleave(jnp.concatenate([edges[:, 1], padi]))

    h0, hk0, st0 = _tc_front(xp, W0, b0.reshape(1, -1), W1, b1.reshape(1, -1),
                             K0, M0)
    w0s, d0s = _sc_gat_layer(hk0, st0, tgt, src)
    h1, hk1, st1 = _tc_mid(w0s[0], w0s[1], d0s[0], d0s[1], h0, K1, M1, S)
    w1s, d1s = _sc_gat_layer(hk1, st1, tgt, src)
    y = _tc_out(w1s[0], w1s[1], d1s[0], d1s[1], h1, Wo, bo.reshape(1, -1), S)
    return y[:N]


# speedup vs baseline: 55.8214x; 1.1162x over previous
import dataclasses
import functools

import jax
import jax.numpy as jnp
from jax import lax
from jax.experimental import pallas as pl
from jax.experimental.pallas import tpu as pltpu
from jax.experimental.pallas import tpu_sc as plsc

N = 10000
E = 320000
D = 128
UNITS = 32
HEADS = 4
HID = UNITS * HEADS
OUT = 2

NC = 2
NS = 16
LANES = 16
NW = NC * NS

N_PAD = 10016
ACCW = 144
CHUNK = 64
E_PAD = 327680
EP_TILE = E_PAD // NW
NCHUNK = EP_TILE // CHUNK
ROWS_PER_TILE = N_PAD // NS

BR = 2504
GRID = N_PAD // BR

_PREC = jax.lax.Precision.DEFAULT


def _dot(a, b):
    return jnp.dot(a, b, precision=_PREC, preferred_element_type=jnp.float32)


def _tc_front(x, W0, b0, W1, b1, K0, M0):
    def body(x_ref, w0_ref, b0_ref, w1_ref, b1_ref, k_ref, m_ref,
             h_ref, hk_ref, st_ref):
        hh = jnp.maximum(_dot(x_ref[...], w0_ref[...]) + b0_ref[...], 0.0)
        hh = jnp.maximum(_dot(hh, w1_ref[...]) + b1_ref[...], 0.0)
        h_ref[...] = hh
        hk = _dot(hh, k_ref[...])
        hk_ref[...] = hk
        st_ref[...] = _dot(hk, m_ref[...])

    full = lambda shape: pl.BlockSpec(shape, lambda i: (0, 0))
    return pl.pallas_call(
        body,
        grid=(GRID,),
        in_specs=[
            pl.BlockSpec((BR, D), lambda i: (i, 0)),
            full((D, HID)), full((1, HID)), full((HID, HID)), full((1, HID)),
            full((HID, HID)), full((HID, 16)),
        ],
        out_specs=[
            pl.BlockSpec((BR, HID), lambda i: (i, 0)),
            pl.BlockSpec((BR, HID), lambda i: (i, 0)),
            pl.BlockSpec((BR, 16), lambda i: (i, 0)),
        ],
        out_shape=[
            jax.ShapeDtypeStruct((N_PAD, HID), jnp.float32),
            jax.ShapeDtypeStruct((N_PAD, HID), jnp.float32),
            jax.ShapeDtypeStruct((N_PAD, 16), jnp.float32),
        ],
    )(x, W0, b0, W1, b1, K0, M0)


def _tc_mid(wA, wB, dA, dB, hprev, K, M, S):
    def body(wa_ref, wb_ref, da_ref, db_ref, hp_ref, k_ref, m_ref, s_ref,
             h_ref, hk_ref, st_ref):
        w = wa_ref[...] + wb_ref[...]
        den = da_ref[...] + db_ref[...]
        den128 = _dot(den, s_ref[...]) + 1e-7
        h1 = jnp.maximum(w / den128, 0.0) + hp_ref[...]
        h_ref[...] = h1
        hk = _dot(h1, k_ref[...])
        hk_ref[...] = hk
        st_ref[...] = _dot(hk, m_ref[...])

    full = lambda shape: pl.BlockSpec(shape, lambda i: (0, 0))
    row = lambda w: pl.BlockSpec((BR, w), lambda i: (i, 0))
    return pl.pallas_call(
        body,
        grid=(GRID,),
        in_specs=[row(HID), row(HID), row(16), row(16), row(HID),
                  full((HID, HID)), full((HID, 16)), full((16, HID))],
        out_specs=[row(HID), row(HID), row(16)],
        out_shape=[
            jax.ShapeDtypeStruct((N_PAD, HID), jnp.float32),
            jax.ShapeDtypeStruct((N_PAD, HID), jnp.float32),
            jax.ShapeDtypeStruct((N_PAD, 16), jnp.float32),
        ],
    )(wA, wB, dA, dB, hprev, K, M, S)


def _tc_out(wA, wB, dA, dB, hprev, Wo, bo, S):
    def body(wa_ref, wb_ref, da_ref, db_ref, hp_ref, wo_ref, bo_ref, s_ref,
             y_ref):
        w = wa_ref[...] + wb_ref[...]
        den = da_ref[...] + db_ref[...]
        den128 = _dot(den, s_ref[...]) + 1e-7
        h2 = jnp.maximum(w / den128, 0.0) + hp_ref[...]
        y_ref[...] = _dot(h2, wo_ref[...]) + bo_ref[...]

    full = lambda shape: pl.BlockSpec(shape, lambda i: (0, 0))
    row = lambda w: pl.BlockSpec((BR, w), lambda i: (i, 0))
    return pl.pallas_call(
        body,
        grid=(GRID,),
        in_specs=[row(HID), row(HID), row(16), row(16), row(HID),
                  full((HID, OUT)), full((1, OUT)), full((16, HID))],
        out_specs=row(OUT),
        out_shape=jax.ShapeDtypeStruct((N_PAD, OUT), jnp.float32),
    )(wA, wB, dA, dB, hprev, Wo, bo, S)


def _sc_gat_layer(hk, sctab, tgt, src):
    mesh = plsc.VectorSubcoreMesh(core_axis_name="c", subcore_axis_name="s")
    cp = pltpu.CompilerParams()
    if "needs_layout_passes" in pltpu.CompilerParams.__dataclass_fields__:
        cp = dataclasses.replace(cp, needs_layout_passes=False)
    if "use_tc_tiling_on_sc" in pltpu.CompilerParams.__dataclass_fields__:
        cp = dataclasses.replace(cp, use_tc_tiling_on_sc=False)

    scratch_types=(
        [pltpu.VMEM((CHUNK,), jnp.int32) for _ in range(4)]
        + [pltpu.VMEM((CHUNK,), jnp.int32) for _ in range(4)]
        + [pltpu.VMEM((CHUNK, 16), jnp.float32) for _ in range(2)]
        + [pltpu.VMEM((CHUNK, 16), jnp.float32) for _ in range(2)]
        + [pltpu.VMEM((CHUNK, HID), jnp.float32) for _ in range(2)]
        + [pltpu.VMEM((CHUNK, ACCW), jnp.float32) for _ in range(2)]
        + [pltpu.VMEM_SHARED((N_PAD, ACCW), jnp.float32)]
        + [pltpu.SemaphoreType.DMA for _ in range(8)]
    )

    @functools.partial(
        pl.kernel,
        out_type=[jax.ShapeDtypeStruct((N_PAD, HID), jnp.float32),
                  jax.ShapeDtypeStruct((N_PAD, HID), jnp.float32),
                  jax.ShapeDtypeStruct((N_PAD, 16), jnp.float32),
                  jax.ShapeDtypeStruct((N_PAD, 16), jnp.float32)],
        mesh=mesh,
        compiler_params=cp,
        scratch_types=scratch_types,
    )
    def k(hk_hbm, sctab_hbm, tgt_hbm, src_hbm, w0_hbm, w1_hbm, d0_hbm, d1_hbm,
          t0, t1, t2, t3, s0, s1, s2, s3, a0, a1, b0, b1, r0, r1, o0, o1,
          acc_sh, i_sem0, i_sem1, i_sem2, i_sem3, gsem0, gsem1, ssem0, ssem1):
        cid = lax.axis_index("c")
        sid = lax.axis_index("s")
        wid = cid * NS + sid
        tbuf, sbuf = (t0, t1, t2, t3), (s0, s1, s2, s3)
        abuf, bbuf, rbuf, obuf = (a0, a1), (b0, b1), (r0, r1), (o0, o1)
        isem = (i_sem0, i_sem1, i_sem2, i_sem3)
        gsem, ssem = (gsem0, gsem1), (ssem0, ssem1)

        zero16 = jnp.zeros((LANES,), jnp.float32)

        for o in obuf:
            @pl.loop(0, CHUNK)
            def _zero_rows(i):
                @pl.loop(0, ACCW, step=LANES)
                def _zero_cols(j):
                    o[i, pl.ds(j, LANES)] = zero16

        row0 = sid * ROWS_PER_TILE
        for off in range(0, ROWS_PER_TILE, CHUNK):
            sz = min(CHUNK, ROWS_PER_TILE - off)
            pltpu.sync_copy(o0.at[pl.ds(0, sz)],
                            acc_sh.at[pl.ds(row0 + off, sz)])
        plsc.subcore_barrier()

        iota = lax.iota(jnp.int32, LANES)
        idiv4 = lax.shift_right_logical(iota, 2)
        imod4 = iota & 3
        col_a = imod4
        col_b = imod4 + HEADS
        col_p = imod4 + HID
        midx = [jnp.full((LANES,), j, jnp.int32) for j in range(LANES)]

        def start_idx(ci, ib):
            base = (ci * NW + wid) * CHUNK
            pltpu.async_copy(tgt_hbm.at[pl.ds(base, CHUNK)], tbuf[ib], isem[ib])
            pltpu.async_copy(src_hbm.at[pl.ds(base, CHUNK)], sbuf[ib], isem[ib])

        def wait_idx(ci, ib):
            base = (ci * NW + wid) * CHUNK
            pltpu.make_async_copy(tgt_hbm.at[pl.ds(base, CHUNK)], tbuf[ib],
                                  isem[ib]).wait()
            pltpu.make_async_copy(src_hbm.at[pl.ds(base, CHUNK)], sbuf[ib],
                                  isem[ib]).wait()

        def start_gathers(gb, ib):
            pltpu.async_copy(sctab_hbm.at[tbuf[ib]], abuf[gb], gsem[gb])
            pltpu.async_copy(sctab_hbm.at[sbuf[ib]], bbuf[gb], gsem[gb])
            pltpu.async_copy(hk_hbm.at[sbuf[ib]], rbuf[gb], gsem[gb])

        def wait_gathers(gb, ib):
            pltpu.make_async_copy(sctab_hbm.at[tbuf[ib]], abuf[gb],
                                  gsem[gb]).wait()
            pltpu.make_async_copy(sctab_hbm.at[sbuf[ib]], bbuf[gb],
                                  gsem[gb]).wait()
            pltpu.make_async_copy(hk_hbm.at[sbuf[ib]], rbuf[gb],
                                  gsem[gb]).wait()

        def wait_scatter(gb, ib):
            pltpu.make_async_copy(obuf[gb], acc_sh.at[tbuf[ib]],
                                  ssem[gb]).wait()

        def compute_chunk(a_v, b_v, rows_v, orow_v):
            @plsc.parallel_loop(0, CHUNK // 4, unroll=4)
            def _group(g):
                e0 = g * 4
                rowidx = idiv4 + e0
                va = plsc.load_gather(a_v, [rowidx, col_a])
                vb = plsc.load_gather(b_v, [rowidx, col_b])
                s = va + vb
                s = jnp.maximum(s, 0.2 * s)
                s = jnp.clip(s, -2.0, 2.0)
                p4 = jnp.exp(s)
                plsc.store_scatter(orow_v, [rowidx, col_p], p4)
                ms = [jnp.take_along_axis(p4, midx[j], axis=0,
                                          mode="promise_in_bounds")
                      for j in range(16)]
                for kk in range(4):
                    e = e0 + kk
                    loads = [rows_v[e, pl.ds(LANES * c, LANES)]
                             for c in range(8)]
                    prods = [loads[c] * ms[4 * kk + c // 2] for c in range(8)]
                    for c in range(8):
                        orow_v[e, pl.ds(LANES * c, LANES)] = prods[c]

        start_idx(0, 0)
        start_idx(1, 1)
        wait_idx(0, 0)
        start_gathers(0, 0)

        @pl.loop(0, NCHUNK, step=4)
        def _quad(ci):
            for b in range(4):
                cur = ci + b
                gb = b % 2

                @pl.when(cur >= 2)
                def _ws():
                    wait_scatter(gb, (b - 2) % 4)

                @pl.when(cur + 2 < NCHUNK)
                def _pi():
                    start_idx(cur + 2, (b + 2) % 4)

                @pl.when(cur + 1 < NCHUNK)
                def _pg():
                    wait_idx(cur + 1, (b + 1) % 4)
                    start_gathers(1 - gb, (b + 1) % 4)

                wait_gathers(gb, b)
                compute_chunk(abuf[gb], bbuf[gb], rbuf[gb], obuf[gb])
                pltpu.async_copy(obuf[gb], acc_sh.at[tbuf[b]], ssem[gb],
                                 add=True)

        wait_scatter(0, 2)
        wait_scatter(1, 3)
        plsc.subcore_barrier()
        for w_hbm, d_hbm, core in ((w0_hbm, d0_hbm, 0), (w1_hbm, d1_hbm, 1)):
            @pl.when(cid == core)
            def _copy_out():
                for off in range(0, ROWS_PER_TILE, CHUNK):
                    sz = min(CHUNK, ROWS_PER_TILE - off)
                    pltpu.sync_copy(
                        acc_sh.at[pl.ds(row0 + off, sz), pl.ds(0, HID)],
                        w_hbm.at[pl.ds(row0 + off, sz)])
                    pltpu.sync_copy(
                        acc_sh.at[pl.ds(row0 + off, sz), pl.ds(HID, 16)],
                        d_hbm.at[pl.ds(row0 + off, sz)])

    return k(hk, sctab, tgt, src)


def _make_M(att_l):
    at_w = att_l[:, :UNITS, 0]
    as_w = att_l[:, UNITS:, 0]
    eye = jnp.eye(HEADS, dtype=jnp.float32)
    Mt = jnp.einsum("hu,hk->huk", at_w, eye).reshape(HID, HEADS)
    Ms = jnp.einsum("hu,hk->huk", as_w, eye).reshape(HID, HEADS)
    return jnp.concatenate([Mt, Ms, jnp.zeros((HID, 8), jnp.float32)], axis=1)


def kernel(x, edges, W0, b0, W1, b1, gat_kernels, gat_attn, Wo, bo):
    K0 = gat_kernels[0].transpose(1, 0, 2).reshape(HID, HID)
    K1 = gat_kernels[1].transpose(1, 0, 2).reshape(HID, HID)
    M0 = _make_M(gat_attn[0])
    M1 = _make_M(gat_attn[1])
    S = jnp.concatenate(
        [jnp.repeat(jnp.eye(HEADS, dtype=jnp.float32), UNITS, axis=1),
         jnp.zeros((12, HID), jnp.float32)], axis=0)

    xp = jnp.zeros((N_PAD, D), jnp.float32).at[:N].set(x)
    padi = N + (jnp.arange(E_PAD - E, dtype=jnp.int32) % (N_PAD - N))
    tgt = jnp.concatenate([edges[:, 0], padi])
    src = jnp.concatenate([edges[:, 1], padi])

    h0, hk0, st0 = _tc_front(xp, W0, b0.reshape(1, -1), W1, b1.reshape(1, -1),
                             K0, M0)
    wa0, wb0, da0, db0 = _sc_gat_layer(hk0, st0, tgt, src)
    h1, hk1, st1 = _tc_mid(wa0, wb0, da0, db0, h0, K1, M1, S)
    wa1, wb1, da1, db1 = _sc_gat_layer(hk1, st1, tgt, src)
    y = _tc_out(wa1, wb1, da1, db1, h1, Wo, bo.reshape(1, -1), S)
    return y[:N]
